# baseline (speedup 1.0000x reference)
import functools

import jax
import jax.numpy as jnp
from jax import lax
from jax.experimental import pallas as pl
from jax.experimental.pallas import tpu as pltpu
from jax.experimental.pallas import tpu_sc as plsc

B = 16384
EMBED_DIM = 32
HIDDEN = 64
CHUNK = 128


def _sc_gather(uid, iid, user_emb, item_emb, nc, nw):
    nchunk = uid.shape[1]
    b_per_w = nchunk * CHUNK
    mesh = plsc.VectorSubcoreMesh(core_axis_name="c", subcore_axis_name="s")

    @functools.partial(
        pl.kernel,
        mesh=mesh,
        compiler_params=pltpu.CompilerParams(use_tc_tiling_on_sc=False),
        out_type=[
            jax.ShapeDtypeStruct((B, EMBED_DIM), jnp.float32),
            jax.ShapeDtypeStruct((B, EMBED_DIM), jnp.float32),
        ],
        scratch_types=[
            pltpu.VMEM((nchunk, CHUNK), jnp.int32),
            pltpu.VMEM((nchunk, CHUNK), jnp.int32),
            pltpu.VMEM((b_per_w, EMBED_DIM), jnp.float32),
            pltpu.VMEM((b_per_w, EMBED_DIM), jnp.float32),
            pltpu.SemaphoreType.DMA,
        ],
    )
    def gather_kernel(uid_h, iid_h, uemb_h, iemb_h, uout_h, iout_h,
                      uidx, iidx, urows, irows, sem):
        wid = lax.axis_index("s") * nc + lax.axis_index("c")
        base = wid * b_per_w
        pltpu.sync_copy(uid_h.at[wid], uidx)
        pltpu.sync_copy(iid_h.at[wid], iidx)
        copies = []
        for j in range(nchunk):
            copies.append(pltpu.async_copy(
                uemb_h.at[uidx.at[j]], urows.at[pl.ds(j * CHUNK, CHUNK)], sem))
            copies.append(pltpu.async_copy(
                iemb_h.at[iidx.at[j]], irows.at[pl.ds(j * CHUNK, CHUNK)], sem))
        for c in copies:
            c.wait()
        pltpu.sync_copy(urows, uout_h.at[pl.ds(base, b_per_w)])
        pltpu.sync_copy(irows, iout_h.at[pl.ds(base, b_per_w)])

    return gather_kernel(uid, iid, user_emb, item_emb)


def _mlp_body(u_ref, v_ref, w1u_ref, w1v_ref, b1_ref, w2_ref, b2_ref, out_ref):
    dn = (((1,), (1,)), ((), ()))
    h = lax.dot_general(u_ref[...], w1u_ref[...], dn,
                        preferred_element_type=jnp.float32)
    h = h + lax.dot_general(v_ref[...], w1v_ref[...], dn,
                            preferred_element_type=jnp.float32)
    h = jnp.maximum(h + b1_ref[...][None, :], 0.0)
    out_ref[...] = jnp.sum(h * w2_ref[...][None, :], axis=1) + b2_ref[...]


def _tc_mlp(u, v, w1u, w1v, b1, w2, b2):
    bm = 2048
    grid = B // bm
    return pl.pallas_call(
        _mlp_body,
        grid=(grid,),
        in_specs=[
            pl.BlockSpec((bm, EMBED_DIM), lambda i: (i, 0)),
            pl.BlockSpec((bm, EMBED_DIM), lambda i: (i, 0)),
            pl.BlockSpec((HIDDEN, EMBED_DIM), lambda i: (0, 0)),
            pl.BlockSpec((HIDDEN, EMBED_DIM), lambda i: (0, 0)),
            pl.BlockSpec((HIDDEN,), lambda i: (0,)),
            pl.BlockSpec((HIDDEN,), lambda i: (0,)),
            pl.BlockSpec((1,), lambda i: (0,)),
        ],
        out_specs=pl.BlockSpec((bm,), lambda i: (i,)),
        out_shape=jax.ShapeDtypeStruct((B,), jnp.float32),
    )(u, v, w1u, w1v, b1, w2, b2)


def kernel(user_ids, item_ids, user_emb, item_emb, W1, b1, W2, b2):
    info = plsc.get_sparse_core_info()
    nc, ns = info.num_cores, info.num_subcores
    nw = nc * ns
    nchunk = B // (nw * CHUNK)
    uid = user_ids.astype(jnp.int32).reshape(nw, nchunk, CHUNK)
    iid = item_ids.astype(jnp.int32).reshape(nw, nchunk, CHUNK)
    u, v = _sc_gather(uid, iid, user_emb, item_emb, nc, nw)
    w1u = W1[:, :EMBED_DIM]
    w1v = W1[:, EMBED_DIM:]
    return _tc_mlp(u, v, w1u, w1v, b1, W2[0], b2)


# speedup vs baseline: 2.2815x; 2.2815x over previous
import functools

import jax
import jax.numpy as jnp
from jax import lax
from jax.experimental import pallas as pl
from jax.experimental.pallas import tpu as pltpu
from jax.experimental.pallas import tpu_sc as plsc

B = 16384
EMBED_DIM = 32
HIDDEN = 64
NCOL = 1_000_000
FULLC = NCOL // 128
TAILN = NCOL - FULLC * 128
NCHUNK = FULLC + 1
SCR = NCHUNK * 32 * 128
CPW = (NCHUNK + 15) // 16


def _sc_linearize(uemb_t, iemb_t, utail, itail, ns):
    mesh = plsc.VectorSubcoreMesh(core_axis_name="c", subcore_axis_name="s")

    @functools.partial(
        pl.kernel,
        mesh=mesh,
        out_type=[
            jax.ShapeDtypeStruct((NCHUNK * 32, 128), jnp.float32),
            jax.ShapeDtypeStruct((NCHUNK * 32, 128), jnp.float32),
        ],
        scratch_types=[
            pltpu.VMEM((32, 128), jnp.float32),
            pltpu.VMEM((32, 128), jnp.float32),
            pltpu.SemaphoreType.DMA,
            pltpu.SemaphoreType.DMA,
            pltpu.SemaphoreType.DMA,
            pltpu.SemaphoreType.DMA,
        ],
    )
    def lin_kernel(uemb_h, iemb_h, utail_h, itail_h, uout_h, iout_h,
                   buf0, buf1, rsem0, rsem1, wsem0, wsem1):
        cid = lax.axis_index("c")
        sid = lax.axis_index("s")

        def do_table(emb_h, tail_h, out2d):
            start = sid * CPW

            def chunk_at(j):
                return jnp.minimum(start + j, NCHUNK - 1)

            def read(c, buf, rsem):
                @pl.when(c < FULLC)
                def _():
                    pltpu.async_copy(
                        emb_h.at[:, pl.ds(c * 128, 128)], buf, rsem)

                @pl.when(c >= FULLC)
                def _():
                    pltpu.async_copy(tail_h, buf, rsem)

            def write(c, buf, wsem):
                pltpu.async_copy(buf, out2d.at[pl.ds(c * 32, 32), :], wsem)

            def wait(sem):
                pltpu.make_async_copy(tail_h, buf0, sem).wait()

            read(chunk_at(0), buf0, rsem0)

            def body(p, _):
                read(chunk_at(2 * p + 1), buf1, rsem1)
                wait(rsem0)
                write(chunk_at(2 * p), buf0, wsem0)
                wait(wsem0)
                read(chunk_at(2 * p + 2), buf0, rsem0)
                wait(rsem1)
                write(chunk_at(2 * p + 1), buf1, wsem1)
                wait(wsem1)
                return 0

            lax.fori_loop(0, CPW // 2, body, 0)
            if CPW % 2 == 1:
                wait(rsem0)
                write(chunk_at(CPW - 1), buf0, wsem0)
                wait(wsem0)

        @pl.when(cid == 0)
        def _():
            do_table(uemb_h, utail_h, uout_h)

        @pl.when(cid == 1)
        def _():
            do_table(iemb_h, itail_h, iout_h)

    return lin_kernel(uemb_t, iemb_t, utail, itail)


def _sc_gather(uid, iid, uscr, iscr, nc, nw):
    b_per_w = B // nw
    nidx = b_per_w * EMBED_DIM
    ndesc = nidx // 128
    mesh = plsc.VectorSubcoreMesh(core_axis_name="c", subcore_axis_name="s")

    @functools.partial(
        pl.kernel,
        mesh=mesh,
        compiler_params=pltpu.CompilerParams(use_tc_tiling_on_sc=False),
        out_type=[
            jax.ShapeDtypeStruct((B * EMBED_DIM,), jnp.float32),
            jax.ShapeDtypeStruct((B * EMBED_DIM,), jnp.float32),
        ],
        scratch_types=[
            pltpu.VMEM((b_per_w,), jnp.int32),
            pltpu.VMEM((b_per_w,), jnp.int32),
            pltpu.VMEM((nidx,), jnp.int32),
            pltpu.VMEM((nidx,), jnp.int32),
            pltpu.VMEM((nidx,), jnp.float32),
            pltpu.VMEM((nidx,), jnp.float32),
            pltpu.SemaphoreType.DMA,
            pltpu.SemaphoreType.DMA,
        ],
    )
    def gather_kernel(uid_h, iid_h, uscr_h, iscr_h, uout_h, iout_h,
                      uids, iids, uidx, iidx, urows, irows, usem, isem):
        wid = lax.axis_index("s") * nc + lax.axis_index("c")
        base = wid * b_per_w
        pltpu.sync_copy(uid_h.at[pl.ds(base, b_per_w)], uids)
        pltpu.sync_copy(iid_h.at[pl.ds(base, b_per_w)], iids)

        dlo = lax.iota(jnp.int32, 16) * 128
        dhi = dlo + 16 * 128

        def idx_body(g, _):
            uvec = uids[pl.ds(g * 16, 16)]
            ubase = ((uvec >> 7) << 12) + (uvec & 127)
            ivec = iids[pl.ds(g * 16, 16)]
            ibase = ((ivec >> 7) << 12) + (ivec & 127)
            for l in range(16):
                j = g * 16 + l
                bu = ubase[l]
                uidx[pl.ds(j * 32, 16)] = bu + dlo
                uidx[pl.ds(j * 32 + 16, 16)] = bu + dhi
                bi = ibase[l]
                iidx[pl.ds(j * 32, 16)] = bi + dlo
                iidx[pl.ds(j * 32 + 16, 16)] = bi + dhi
            return 0

        lax.fori_loop(0, b_per_w // 16, idx_body, 0)

        def fire(g, _):
            pltpu.async_copy(uscr_h.at[uidx.at[pl.ds(g * 128, 128)]],
                             urows.at[pl.ds(g * 128, 128)], usem)
            pltpu.async_copy(iscr_h.at[iidx.at[pl.ds(g * 128, 128)]],
                             irows.at[pl.ds(g * 128, 128)], isem)
            return 0

        lax.fori_loop(0, ndesc, fire, 0)

        def drain(g, _):
            pltpu.make_async_copy(uscr_h.at[uidx.at[pl.ds(0, 128)]],
                                  urows.at[pl.ds(0, 128)], usem).wait()
            pltpu.make_async_copy(iscr_h.at[iidx.at[pl.ds(0, 128)]],
                                  irows.at[pl.ds(0, 128)], isem).wait()
            return 0

        lax.fori_loop(0, ndesc, drain, 0)

        pltpu.sync_copy(urows, uout_h.at[pl.ds(base * EMBED_DIM, nidx)])
        pltpu.sync_copy(irows, iout_h.at[pl.ds(base * EMBED_DIM, nidx)])

    return gather_kernel(uid, iid, uscr, iscr)


def _mlp_body(u_ref, v_ref, w1u_ref, w1v_ref, b1_ref, w2_ref, b2_ref, out_ref):
    dn = (((1,), (1,)), ((), ()))
    h = lax.dot_general(u_ref[...], w1u_ref[...], dn,
                        preferred_element_type=jnp.float32)
    h = h + lax.dot_general(v_ref[...], w1v_ref[...], dn,
                            preferred_element_type=jnp.float32)
    h = jnp.maximum(h + b1_ref[...][None, :], 0.0)
    out_ref[...] = jnp.sum(h * w2_ref[...][None, :], axis=1) + b2_ref[...]


def _tc_mlp(u, v, w1u, w1v, b1, w2, b2):
    bm = 2048
    grid = B // bm
    return pl.pallas_call(
        _mlp_body,
        grid=(grid,),
        in_specs=[
            pl.BlockSpec((bm, EMBED_DIM), lambda i: (i, 0)),
            pl.BlockSpec((bm, EMBED_DIM), lambda i: (i, 0)),
            pl.BlockSpec((HIDDEN, EMBED_DIM), lambda i: (0, 0)),
            pl.BlockSpec((HIDDEN, EMBED_DIM), lambda i: (0, 0)),
            pl.BlockSpec((HIDDEN,), lambda i: (0,)),
            pl.BlockSpec((HIDDEN,), lambda i: (0,)),
            pl.BlockSpec((1,), lambda i: (0,)),
        ],
        out_specs=pl.BlockSpec((bm,), lambda i: (i,)),
        out_shape=jax.ShapeDtypeStruct((B,), jnp.float32),
    )(u, v, w1u, w1v, b1, w2, b2)


def kernel(user_ids, item_ids, user_emb, item_emb, W1, b1, W2, b2):
    info = plsc.get_sparse_core_info()
    nc, ns = info.num_cores, info.num_subcores
    nw = nc * ns
    uid = user_ids.astype(jnp.int32)
    iid = item_ids.astype(jnp.int32)
    utail = jnp.pad(user_emb[FULLC * 128:, :].T, ((0, 0), (0, 128 - TAILN)))
    itail = jnp.pad(item_emb[FULLC * 128:, :].T, ((0, 0), (0, 128 - TAILN)))
    uscr, iscr = _sc_linearize(user_emb.T, item_emb.T, utail, itail, ns)
    u1d, v1d = _sc_gather(uid, iid, uscr.reshape(SCR), iscr.reshape(SCR),
                          nc, nw)
    u = u1d.reshape(B, EMBED_DIM)
    v = v1d.reshape(B, EMBED_DIM)
    w1u = W1[:, :EMBED_DIM]
    w1v = W1[:, EMBED_DIM:]
    return _tc_mlp(u, v, w1u, w1v, b1, W2[0], b2)


# speedup vs baseline: 3.3099x; 1.4507x over previous
import functools

import jax
import jax.numpy as jnp
from jax import lax
from jax.experimental import pallas as pl
from jax.experimental.pallas import tpu as pltpu
from jax.experimental.pallas import tpu_sc as plsc

B = 16384
EMBED_DIM = 32
HIDDEN = 64
NCOL = 1_000_000
RW = 1024
FULLC = NCOL // RW
TAILN = NCOL - FULLC * RW
NCHUNK = FULLC + 1
SCR = NCHUNK * 32 * RW
CPW = (NCHUNK + 15) // 16


def _sc_linearize(uemb_t, iemb_t, utail, itail, ns):
    mesh = plsc.VectorSubcoreMesh(core_axis_name="c", subcore_axis_name="s")

    @functools.partial(
        pl.kernel,
        mesh=mesh,
        out_type=[
            jax.ShapeDtypeStruct((SCR // 128, 128), jnp.float32),
            jax.ShapeDtypeStruct((SCR // 128, 128), jnp.float32),
        ],
        scratch_types=[
            pltpu.VMEM((32, RW), jnp.float32),
            pltpu.VMEM((32, RW), jnp.float32),
            pltpu.SemaphoreType.DMA,
            pltpu.SemaphoreType.DMA,
            pltpu.SemaphoreType.DMA,
            pltpu.SemaphoreType.DMA,
        ],
    )
    def lin_kernel(uemb_h, iemb_h, utail_h, itail_h, uout_h, iout_h,
                   buf0, buf1, rsem0, rsem1, wsem0, wsem1):
        cid = lax.axis_index("c")
        sid = lax.axis_index("s")

        def do_table(emb_h, tail_h, out2d):
            start = sid * CPW

            def chunk_at(j):
                return jnp.minimum(start + j, NCHUNK - 1)

            def read(c, buf, rsem):
                @pl.when(c < FULLC)
                def _():
                    pltpu.async_copy(
                        emb_h.at[:, pl.ds(c * RW, RW)], buf, rsem)

                @pl.when(c >= FULLC)
                def _():
                    pltpu.async_copy(tail_h, buf, rsem)

            def write(c, buf, wsem):
                for k in range(RW // 128):
                    pltpu.async_copy(
                        buf.at[:, pl.ds(k * 128, 128)],
                        out2d.at[pl.ds((c * (RW // 128) + k) * 32, 32), :],
                        wsem)

            def wait_r(sem):
                pltpu.make_async_copy(tail_h, buf0, sem).wait()

            def wait_w(sem):
                for _ in range(RW // 128):
                    pltpu.make_async_copy(
                        buf0.at[:, pl.ds(0, 128)],
                        out2d.at[pl.ds(0, 32), :], sem).wait()

            read(chunk_at(0), buf0, rsem0)

            def body(p, _):
                read(chunk_at(2 * p + 1), buf1, rsem1)
                wait_r(rsem0)
                write(chunk_at(2 * p), buf0, wsem0)
                wait_w(wsem0)

                @pl.when(2 * p + 2 < CPW)
                def _():
                    read(chunk_at(2 * p + 2), buf0, rsem0)

                wait_r(rsem1)
                write(chunk_at(2 * p + 1), buf1, wsem1)
                wait_w(wsem1)
                return 0

            lax.fori_loop(0, CPW // 2, body, 0)
            if CPW % 2 == 1:
                wait_r(rsem0)
                write(chunk_at(CPW - 1), buf0, wsem0)
                wait_w(wsem0)

        @pl.when(cid == 0)
        def _():
            do_table(uemb_h, utail_h, uout_h)

        @pl.when(cid == 1)
        def _():
            do_table(iemb_h, itail_h, iout_h)

    return lin_kernel(uemb_t, iemb_t, utail, itail)


def _sc_gather(uid, iid, uscr, iscr, nc, nw):
    b_per_w = B // nw
    nidx = b_per_w * EMBED_DIM
    ndesc = nidx // 128
    mesh = plsc.VectorSubcoreMesh(core_axis_name="c", subcore_axis_name="s")

    @functools.partial(
        pl.kernel,
        mesh=mesh,
        compiler_params=pltpu.CompilerParams(use_tc_tiling_on_sc=False),
        out_type=[
            jax.ShapeDtypeStruct((B * EMBED_DIM,), jnp.float32),
            jax.ShapeDtypeStruct((B * EMBED_DIM,), jnp.float32),
        ],
        scratch_types=[
            pltpu.VMEM((b_per_w,), jnp.int32),
            pltpu.VMEM((b_per_w,), jnp.int32),
            pltpu.VMEM((nidx,), jnp.int32),
            pltpu.VMEM((nidx,), jnp.int32),
            pltpu.VMEM((nidx,), jnp.float32),
            pltpu.VMEM((nidx,), jnp.float32),
            pltpu.SemaphoreType.DMA,
            pltpu.SemaphoreType.DMA,
        ],
    )
    def gather_kernel(uid_h, iid_h, uscr_h, iscr_h, uout_h, iout_h,
                      uids, iids, uidx, iidx, urows, irows, usem, isem):
        wid = lax.axis_index("s") * nc + lax.axis_index("c")
        base = wid * b_per_w
        pltpu.sync_copy(uid_h.at[pl.ds(base, b_per_w)], uids)
        pltpu.sync_copy(iid_h.at[pl.ds(base, b_per_w)], iids)

        dlo = lax.iota(jnp.int32, 16) * 128
        dhi = dlo + 16 * 128

        def idx_body(g, _):
            uvec = uids[pl.ds(g * 16, 16)]
            ubase = ((uvec >> 7) << 12) + (uvec & 127)
            ivec = iids[pl.ds(g * 16, 16)]
            ibase = ((ivec >> 7) << 12) + (ivec & 127)
            for l in range(16):
                j = g * 16 + l
                bu = ubase[l]
                uidx[pl.ds(j * 32, 16)] = bu + dlo
                uidx[pl.ds(j * 32 + 16, 16)] = bu + dhi
                bi = ibase[l]
                iidx[pl.ds(j * 32, 16)] = bi + dlo
                iidx[pl.ds(j * 32 + 16, 16)] = bi + dhi
            return 0

        lax.fori_loop(0, b_per_w // 16, idx_body, 0)

        def fire(g, _):
            pltpu.async_copy(uscr_h.at[uidx.at[pl.ds(g * 128, 128)]],
                             urows.at[pl.ds(g * 128, 128)], usem)
            pltpu.async_copy(iscr_h.at[iidx.at[pl.ds(g * 128, 128)]],
                             irows.at[pl.ds(g * 128, 128)], isem)
            return 0

        lax.fori_loop(0, ndesc, fire, 0)

        def drain(g, _):
            pltpu.make_async_copy(uscr_h.at[uidx.at[pl.ds(0, 128)]],
                                  urows.at[pl.ds(0, 128)], usem).wait()
            pltpu.make_async_copy(iscr_h.at[iidx.at[pl.ds(0, 128)]],
                                  irows.at[pl.ds(0, 128)], isem).wait()
            return 0

        lax.fori_loop(0, ndesc, drain, 0)

        pltpu.sync_copy(urows, uout_h.at[pl.ds(base * EMBED_DIM, nidx)])
        pltpu.sync_copy(irows, iout_h.at[pl.ds(base * EMBED_DIM, nidx)])

    return gather_kernel(uid, iid, uscr, iscr)


def _mlp_body(u_ref, v_ref, w1u_ref, w1v_ref, b1_ref, w2_ref, b2_ref, out_ref):
    dn = (((1,), (1,)), ((), ()))
    h = lax.dot_general(u_ref[...], w1u_ref[...], dn,
                        preferred_element_type=jnp.float32)
    h = h + lax.dot_general(v_ref[...], w1v_ref[...], dn,
                            preferred_element_type=jnp.float32)
    h = jnp.maximum(h + b1_ref[...][None, :], 0.0)
    out_ref[...] = jnp.sum(h * w2_ref[...][None, :], axis=1) + b2_ref[...]


def _tc_mlp(u, v, w1u, w1v, b1, w2, b2):
    bm = 2048
    grid = B // bm
    return pl.pallas_call(
        _mlp_body,
        grid=(grid,),
        in_specs=[
            pl.BlockSpec((bm, EMBED_DIM), lambda i: (i, 0)),
            pl.BlockSpec((bm, EMBED_DIM), lambda i: (i, 0)),
            pl.BlockSpec((HIDDEN, EMBED_DIM), lambda i: (0, 0)),
            pl.BlockSpec((HIDDEN, EMBED_DIM), lambda i: (0, 0)),
            pl.BlockSpec((HIDDEN,), lambda i: (0,)),
            pl.BlockSpec((HIDDEN,), lambda i: (0,)),
            pl.BlockSpec((1,), lambda i: (0,)),
        ],
        out_specs=pl.BlockSpec((bm,), lambda i: (i,)),
        out_shape=jax.ShapeDtypeStruct((B,), jnp.float32),
    )(u, v, w1u, w1v, b1, w2, b2)


def kernel(user_ids, item_ids, user_emb, item_emb, W1, b1, W2, b2):
    info = plsc.get_sparse_core_info()
    nc, ns = info.num_cores, info.num_subcores
    nw = nc * ns
    uid = user_ids.astype(jnp.int32)
    iid = item_ids.astype(jnp.int32)
    utail = jnp.pad(user_emb[FULLC * RW:, :].T, ((0, 0), (0, RW - TAILN)))
    itail = jnp.pad(item_emb[FULLC * RW:, :].T, ((0, 0), (0, RW - TAILN)))
    uscr, iscr = _sc_linearize(user_emb.T, item_emb.T, utail, itail, ns)
    u1d, v1d = _sc_gather(uid, iid, uscr.reshape(SCR), iscr.reshape(SCR),
                          nc, nw)
    u = u1d.reshape(B, EMBED_DIM)
    v = v1d.reshape(B, EMBED_DIM)
    w1u = W1[:, :EMBED_DIM]
    w1v = W1[:, EMBED_DIM:]
    return _tc_mlp(u, v, w1u, w1v, b1, W2[0], b2)
